# baseline (speedup 1.0000x reference)
import functools

import jax
import jax.numpy as jnp
from jax import lax
from jax.experimental import pallas as pl
from jax.experimental.pallas import tpu as pltpu
from jax.experimental.pallas import tpu_sc as plsc

NUM_RELATIONS = 1000000
EMB_D = 32
BATCH_B = 4096
FIELDS_F = 26
TOTAL = BATCH_B * FIELDS_F
PACK = 128 // EMB_D
TABLE_ROWS = NUM_RELATIONS // PACK
OUT_ROWS = TOTAL // PACK

_INFO = plsc.get_sparse_core_info()
NC = _INFO.num_cores
NS = _INFO.num_subcores
NW = NC * NS
B_PER_W = TOTAL // NW
CHUNK = 128
NCHUNK = B_PER_W // CHUNK
ORPC = CHUNK // PACK
NPAIR = NCHUNK // 2

_MESH = plsc.VectorSubcoreMesh(core_axis_name="c", subcore_axis_name="s")


@functools.partial(
    pl.kernel,
    mesh=_MESH,
    out_type=jax.ShapeDtypeStruct((OUT_ROWS, 128), jnp.float32),
    scratch_types=[
        pltpu.VMEM((B_PER_W,), jnp.int32),
        pltpu.VMEM((B_PER_W,), jnp.int32),
        pltpu.VMEM((CHUNK, 128), jnp.float32),
        pltpu.VMEM((CHUNK, 128), jnp.float32),
        pltpu.VMEM((ORPC, 128), jnp.float32),
        pltpu.VMEM((ORPC, 128), jnp.float32),
        pltpu.SemaphoreType.DMA,
        pltpu.SemaphoreType.DMA,
        pltpu.SemaphoreType.DMA,
        pltpu.SemaphoreType.DMA,
    ],
)
def _lookup(idx_hbm, table_hbm, out_hbm, idx_v, qid_v, p_a, p_b, o_a, o_b,
            sg_a, sg_b, sw_a, sw_b):
    wid = lax.axis_index("s") * NC + lax.axis_index("c")
    ibase = wid * B_PER_W
    obase = wid * (NCHUNK * ORPC)
    lane = lax.iota(jnp.int32, 16)

    pltpu.sync_copy(idx_hbm.at[pl.ds(ibase, B_PER_W)], idx_v)

    def _qid(i, _):
        v = idx_v[pl.ds(i * 16, 16)]
        qid_v[pl.ds(i * 16, 16)] = lax.shift_right_logical(v, 2)
        return _

    lax.fori_loop(0, B_PER_W // 16, _qid, None)

    def _gather(j, pbuf, sem):
        return pltpu.async_copy(
            table_hbm.at[qid_v.at[pl.ds(j * CHUNK, CHUNK)]], pbuf, sem)

    def _drain_gather(pbuf, sem):
        pltpu.make_async_copy(table_hbm.at[pl.ds(0, CHUNK)], pbuf, sem).wait()

    def _drain_write(obuf, sem):
        pltpu.make_async_copy(out_hbm.at[pl.ds(0, ORPC)], obuf, sem).wait()

    def _extract(j, pbuf, obuf):
        jbase = j * CHUNK
        for g in range(CHUNK // 16):
            idx16 = idx_v[pl.ds(jbase + g * 16, 16)]
            off16 = lax.shift_left(jnp.bitwise_and(idx16, 3), 5)
            for l in range(16):
                k = g * 16 + l
                off = off16[l]
                orow, ocol = k // PACK, (k % PACK) * EMB_D
                obuf[orow, pl.ds(ocol, 16)] = pbuf[k, pl.ds(off, 16)]
                obuf[orow, pl.ds(ocol + 16, 16)] = pbuf[k, pl.ds(off + 16, 16)]

    def _writeback(j, obuf, sem):
        pltpu.async_copy(obuf, out_hbm.at[pl.ds(obase + j * ORPC, ORPC)], sem)

    _gather(0, p_a, sg_a)

    def _pair(jj, _):
        j0 = jj * 2
        j1 = j0 + 1
        _gather(j1, p_b, sg_b)
        _drain_gather(p_a, sg_a)

        @pl.when(jj > 0)
        def _():
            _drain_write(o_a, sw_a)

        _extract(j0, p_a, o_a)
        _writeback(j0, o_a, sw_a)

        @pl.when(jj < NPAIR - 1)
        def _():
            _gather(j0 + 2, p_a, sg_a)

        _drain_gather(p_b, sg_b)

        @pl.when(jj > 0)
        def _():
            _drain_write(o_b, sw_b)

        _extract(j1, p_b, o_b)
        _writeback(j1, o_b, sw_b)
        return _

    lax.fori_loop(0, NPAIR, _pair, None)
    _drain_write(o_a, sw_a)
    _drain_write(o_b, sw_b)


def kernel(relation_ids, embedding_table):
    ids = relation_ids.astype(jnp.int32).reshape(TOTAL)
    table = embedding_table.reshape(TABLE_ROWS, 128)
    out = _lookup(ids, table)
    return out.reshape(BATCH_B, FIELDS_F, EMB_D)
